# baseline (device time: 122683 ns/iter reference)
import os

import jax
import jax.numpy as jnp
from jax import lax
from jax.experimental import pallas as pl
from jax.experimental.pallas import tpu as pltpu

_NO_RING = os.environ.get("NO_RING") == "1"

N_DEV = 32
SQ = 1024
H = 8
DH = 128
WIN = 128
SCALE = 0.08838834764831843


def kernel(x, Wq, K_ext, V_ext, Wo):
    idx = lax.axis_index("i")
    k_loc = jnp.transpose(
        lax.dynamic_slice_in_dim(K_ext[0], idx * H, H, axis=1), (1, 0, 2)
    )
    v_loc = jnp.transpose(
        lax.dynamic_slice_in_dim(V_ext[0], idx * H, H, axis=1), (1, 0, 2)
    )
    x2 = x[0]

    def body(x_ref, wq_ref, k_ref, v_ref, wo_ref, out_ref,
             acc_ref, ctx_ref, sbuf, gat, xbuf, ybuf, zbuf,
             send_sems, recv_sems):
        i = lax.axis_index("i")
        mz = i // 8
        p = i % 8
        my = p // 2
        b = p % 2
        mx = jnp.where(my % 2 == 0, b, 1 - b)

        def lid(cx, cy, cz):
            return 8 * cz + 2 * cy + jnp.where(cy % 2 == 0, cx, 1 - cx)

        px = lid(1 - mx, my, mz)
        ry = lid(mx, (my + 1) % 4, mz)
        ly = lid(mx, (my + 3) % 4, mz)
        rz = (i + 8) % N_DEV
        lz = (i + 24) % N_DEV

        barrier_sem = pltpu.get_barrier_semaphore()
        for nbr in (px, ry, ly, rz, lz):
            pl.semaphore_signal(
                barrier_sem, inc=1,
                device_id=(nbr,), device_id_type=pl.DeviceIdType.MESH,
            )
        pl.semaphore_wait(barrier_sem, 5)

        QB = 256
        KW = 512
        for h in range(H):
            q = jnp.dot(
                x_ref[...], wq_ref[:, h * DH:(h + 1) * DH],
                preferred_element_type=jnp.float32,
            )
            for qb in range(SQ // QB):
                qs = qb * QB
                ks = min(max(qs - WIN, 0), SQ - KW)
                s = lax.dot_general(
                    q[qs:qs + QB], k_ref[h, ks:ks + KW],
                    (((1,), (1,)), ((), ())),
                    preferred_element_type=jnp.float32,
                ) * SCALE
                r_io = lax.broadcasted_iota(jnp.int32, (QB, KW), 0)
                c_io = lax.broadcasted_iota(jnp.int32, (QB, KW), 1)
                mask = jnp.abs((qs + r_io) - (ks + c_io)) <= WIN
                s = jnp.where(mask, s, -1e9)
                m = jnp.max(s, axis=-1, keepdims=True)
                w = jnp.exp(s - m)
                w = w / jnp.sum(w, axis=-1, keepdims=True)
                ctx_ref[qs:qs + QB, h * DH:(h + 1) * DH] = jnp.dot(
                    w, v_ref[h, ks:ks + KW],
                    preferred_element_type=jnp.float32,
                )
        acc_ref[...] = jnp.dot(
            ctx_ref[...], wo_ref[...], preferred_element_type=jnp.float32
        )

        if _NO_RING:
            out_ref[0] = acc_ref[...]
            return

        def send(st, src, dst, target):
            rdma = pltpu.make_async_remote_copy(
                src_ref=src, dst_ref=dst,
                send_sem=send_sems.at[st], recv_sem=recv_sems.at[st],
                device_id=(target,), device_id_type=pl.DeviceIdType.MESH,
            )
            rdma.start()
            rdma.wait()


        half = 512
        sbuf[...] = acc_ref[pl.ds((1 - mx) * half, half)].astype(jnp.bfloat16)
        send(0, sbuf, xbuf, px)
        acc_ref[pl.ds(mx * half, half)] = (
            acc_ref[pl.ds(mx * half, half)] + xbuf[...].astype(jnp.float32)
        )

        xb = mx * half
        for st in range(3):
            c_send = (my - st) % 4
            c_recv = (my - st - 1) % 4
            sbuf[pl.ds(0, 128)] = (
                acc_ref[pl.ds(xb + c_send * 128, 128)].astype(jnp.bfloat16)
            )
            send(1 + st, sbuf.at[pl.ds(0, 128)], ybuf.at[st], ry)
            acc_ref[pl.ds(xb + c_recv * 128, 128)] = (
                acc_ref[pl.ds(xb + c_recv * 128, 128)]
                + ybuf[st].astype(jnp.float32)
            )
        r_y = (my + 1) % 4

        yb = xb + r_y * 128
        for st in range(3):
            c_send = (mz - st) % 4
            c_recv = (mz - st - 1) % 4
            sbuf[pl.ds(0, 32)] = (
                acc_ref[pl.ds(yb + c_send * 32, 32)].astype(jnp.bfloat16)
            )
            send(4 + st, sbuf.at[pl.ds(0, 32)], zbuf.at[st], rz)
            acc_ref[pl.ds(yb + c_recv * 32, 32)] = (
                acc_ref[pl.ds(yb + c_recv * 32, 32)]
                + zbuf[st].astype(jnp.float32)
            )
        r_z = (mz + 1) % 4

        own = yb + r_z * 32
        gat[pl.ds(own, 32)] = acc_ref[pl.ds(own, 32)].astype(jnp.bfloat16)

        for st in range(3):
            c_send = (mz + 1 - st) % 4
            send(7 + st,
                 gat.at[pl.ds(yb + c_send * 32, 32)],
                 gat.at[pl.ds(yb + c_send * 32, 32)], rz)

        for st in range(3):
            c_send = (my + 1 - st) % 4
            send(10 + st,
                 gat.at[pl.ds(xb + c_send * 128, 128)],
                 gat.at[pl.ds(xb + c_send * 128, 128)], ry)

        send(13, gat.at[pl.ds(xb, half)], gat.at[pl.ds(xb, half)], px)

        out_ref[0] = gat[...].astype(jnp.float32)

    out = pl.pallas_call(
        body,
        out_shape=jax.ShapeDtypeStruct((1, SQ, SQ), jnp.float32),
        in_specs=[pl.BlockSpec(memory_space=pltpu.VMEM)] * 5,
        out_specs=pl.BlockSpec(memory_space=pltpu.VMEM),
        scratch_shapes=[
            pltpu.VMEM((SQ, SQ), jnp.float32),
            pltpu.VMEM((SQ, H * DH), jnp.float32),
            pltpu.VMEM((512, SQ), jnp.bfloat16),
            pltpu.VMEM((SQ, SQ), jnp.bfloat16),
            pltpu.VMEM((512, SQ), jnp.bfloat16),
            pltpu.VMEM((3, 128, SQ), jnp.bfloat16),
            pltpu.VMEM((3, 32, SQ), jnp.bfloat16),
            pltpu.SemaphoreType.DMA((14,)),
            pltpu.SemaphoreType.DMA((14,)),
        ],
        compiler_params=pltpu.CompilerParams(
            collective_id=0,
            vmem_limit_bytes=100 * 1024 * 1024,
        ),
    )(x2, Wq, k_loc, v_loc, Wo)
    return out


# device time: 112348 ns/iter; 1.0920x vs baseline; 1.0920x over previous
import os

import jax
import jax.numpy as jnp
from jax import lax
from jax.experimental import pallas as pl
from jax.experimental.pallas import tpu as pltpu

_NO_RING = os.environ.get("NO_RING") == "1"

N_DEV = 32
SQ = 1024
H = 8
DH = 128
WIN = 128
SCALE = 0.08838834764831843


def kernel(x, Wq, K_ext, V_ext, Wo):
    idx = lax.axis_index("i")
    k_loc = jnp.transpose(
        lax.dynamic_slice_in_dim(K_ext[0], idx * H, H, axis=1), (1, 0, 2)
    )
    v_loc = jnp.transpose(
        lax.dynamic_slice_in_dim(V_ext[0], idx * H, H, axis=1), (1, 0, 2)
    )
    x2 = x[0]

    def body(x_ref, wq_ref, k_ref, v_ref, wo_ref, out_ref,
             acc_ref, ctx_ref, sbuf, gat, xbuf, ybuf, zbuf,
             send_sems, recv_sems):
        i = lax.axis_index("i")
        mz = i // 8
        p = i % 8
        my = p // 2
        b = p % 2
        mx = jnp.where(my % 2 == 0, b, 1 - b)

        def lid(cx, cy, cz):
            return 8 * cz + 2 * cy + jnp.where(cy % 2 == 0, cx, 1 - cx)

        px = lid(1 - mx, my, mz)
        ry = lid(mx, (my + 1) % 4, mz)
        ly = lid(mx, (my + 3) % 4, mz)
        rz = (i + 8) % N_DEV
        lz = (i + 24) % N_DEV

        barrier_sem = pltpu.get_barrier_semaphore()
        for nbr in (px, ry, ly, rz, lz):
            pl.semaphore_signal(
                barrier_sem, inc=1,
                device_id=(nbr,), device_id_type=pl.DeviceIdType.MESH,
            )
        pl.semaphore_wait(barrier_sem, 5)

        QB = 256
        KW = 512
        half = 512

        def compute_half(base):
            for h in range(H):
                qh = jnp.dot(
                    x_ref[pl.ds(base, half)], wq_ref[:, h * DH:(h + 1) * DH],
                    preferred_element_type=jnp.float32,
                )
                for qb in range(half // QB):
                    qs = base + qb * QB
                    ks = jnp.clip(qs - WIN, 0, SQ - KW)
                    s = lax.dot_general(
                        qh[qb * QB:(qb + 1) * QB], k_ref[h, pl.ds(ks, KW)],
                        (((1,), (1,)), ((), ())),
                        preferred_element_type=jnp.float32,
                    ) * SCALE
                    r_io = lax.broadcasted_iota(jnp.int32, (QB, KW), 0)
                    c_io = lax.broadcasted_iota(jnp.int32, (QB, KW), 1)
                    mask = jnp.abs((qs + r_io) - (ks + c_io)) <= WIN
                    s = jnp.where(mask, s, -1e9)
                    m = jnp.max(s, axis=-1, keepdims=True)
                    w = jnp.exp(s - m)
                    w = w / jnp.sum(w, axis=-1, keepdims=True)
                    ctx_ref[pl.ds(qs, QB), h * DH:(h + 1) * DH] = jnp.dot(
                        w, v_ref[h, pl.ds(ks, KW)],
                        preferred_element_type=jnp.float32,
                    )
            acc_ref[pl.ds(base, half)] = jnp.dot(
                ctx_ref[pl.ds(base, half)], wo_ref[...],
                preferred_element_type=jnp.float32,
            )

        other = (1 - mx) * half
        mine = mx * half

        if _NO_RING:
            compute_half(other)
            compute_half(mine)
            out_ref[0] = acc_ref[...]
            return

        def send(st, src, dst, target):
            rdma = pltpu.make_async_remote_copy(
                src_ref=src, dst_ref=dst,
                send_sem=send_sems.at[st], recv_sem=recv_sems.at[st],
                device_id=(target,), device_id_type=pl.DeviceIdType.MESH,
            )
            rdma.start()
            rdma.wait()


        compute_half(other)
        sbuf[...] = acc_ref[pl.ds(other, half)].astype(jnp.bfloat16)
        x_rdma = pltpu.make_async_remote_copy(
            src_ref=sbuf, dst_ref=xbuf,
            send_sem=send_sems.at[0], recv_sem=recv_sems.at[0],
            device_id=(px,), device_id_type=pl.DeviceIdType.MESH,
        )
        x_rdma.start()
        compute_half(mine)
        x_rdma.wait()
        acc_ref[pl.ds(mx * half, half)] = (
            acc_ref[pl.ds(mx * half, half)] + xbuf[...].astype(jnp.float32)
        )

        xb = mx * half
        for st in range(3):
            c_send = (my - st) % 4
            c_recv = (my - st - 1) % 4
            sbuf[pl.ds(0, 128)] = (
                acc_ref[pl.ds(xb + c_send * 128, 128)].astype(jnp.bfloat16)
            )
            send(1 + st, sbuf.at[pl.ds(0, 128)], ybuf.at[st], ry)
            acc_ref[pl.ds(xb + c_recv * 128, 128)] = (
                acc_ref[pl.ds(xb + c_recv * 128, 128)]
                + ybuf[st].astype(jnp.float32)
            )
        r_y = (my + 1) % 4

        yb = xb + r_y * 128
        for st in range(3):
            c_send = (mz - st) % 4
            c_recv = (mz - st - 1) % 4
            sbuf[pl.ds(0, 32)] = (
                acc_ref[pl.ds(yb + c_send * 32, 32)].astype(jnp.bfloat16)
            )
            send(4 + st, sbuf.at[pl.ds(0, 32)], zbuf.at[st], rz)
            acc_ref[pl.ds(yb + c_recv * 32, 32)] = (
                acc_ref[pl.ds(yb + c_recv * 32, 32)]
                + zbuf[st].astype(jnp.float32)
            )
        r_z = (mz + 1) % 4

        own = yb + r_z * 32
        gat[pl.ds(own, 32)] = acc_ref[pl.ds(own, 32)].astype(jnp.bfloat16)

        for st in range(3):
            c_send = (mz + 1 - st) % 4
            send(7 + st,
                 gat.at[pl.ds(yb + c_send * 32, 32)],
                 gat.at[pl.ds(yb + c_send * 32, 32)], rz)

        for st in range(3):
            c_send = (my + 1 - st) % 4
            send(10 + st,
                 gat.at[pl.ds(xb + c_send * 128, 128)],
                 gat.at[pl.ds(xb + c_send * 128, 128)], ry)

        send(13, gat.at[pl.ds(xb, half)], gat.at[pl.ds(xb, half)], px)

        out_ref[0] = gat[...].astype(jnp.float32)

    out = pl.pallas_call(
        body,
        out_shape=jax.ShapeDtypeStruct((1, SQ, SQ), jnp.float32),
        in_specs=[pl.BlockSpec(memory_space=pltpu.VMEM)] * 5,
        out_specs=pl.BlockSpec(memory_space=pltpu.VMEM),
        scratch_shapes=[
            pltpu.VMEM((SQ, SQ), jnp.float32),
            pltpu.VMEM((SQ, H * DH), jnp.float32),
            pltpu.VMEM((512, SQ), jnp.bfloat16),
            pltpu.VMEM((SQ, SQ), jnp.bfloat16),
            pltpu.VMEM((512, SQ), jnp.bfloat16),
            pltpu.VMEM((3, 128, SQ), jnp.bfloat16),
            pltpu.VMEM((3, 32, SQ), jnp.bfloat16),
            pltpu.SemaphoreType.DMA((14,)),
            pltpu.SemaphoreType.DMA((14,)),
        ],
        compiler_params=pltpu.CompilerParams(
            collective_id=0,
            vmem_limit_bytes=100 * 1024 * 1024,
        ),
    )(x2, Wq, k_loc, v_loc, Wo)
    return out
